# baseline (speedup 1.0000x reference)
import functools
import jax
import jax.numpy as jnp
from jax import lax
from jax.experimental import pallas as pl
from jax.experimental.pallas import tpu as pltpu
from jax.experimental.pallas import tpu_sc as plsc


def _make_sc_gather(total_rows, n_rows_x, n_rows_y):
    info = plsc.get_sparse_core_info()
    NC, NS, L = info.num_cores, info.num_subcores, info.num_lanes
    NW = NC * NS
    R = total_rows // NW
    W = R // 2
    CR = 128
    G = R // CR
    SB = 2
    S = G // SB
    NB = 6
    MAGIC = jnp.float32(1.5 * (2 ** 23))

    mesh = plsc.VectorSubcoreMesh(core_axis_name="c", subcore_axis_name="s")

    @functools.partial(
        pl.kernel,
        mesh=mesh,
        compiler_params=pltpu.CompilerParams(use_tc_tiling_on_sc=False,
                                             needs_layout_passes=False),
        out_type=jax.ShapeDtypeStruct((total_rows, 64), jnp.float32),
        scratch_types=[
            pltpu.VMEM_SHARED((1024, 64), jnp.float32),
            pltpu.VMEM((W,), jnp.float32),
            pltpu.VMEM((W,), jnp.float32),
            pltpu.VMEM((NB, SB, CR), jnp.int32),
        ] + [pltpu.VMEM((SB * CR, 64), jnp.float32)] * NB
          + [pltpu.SemaphoreType.DMA] * (2 * NB),
    )
    def k(table_hbm, cxy_hbm, out_hbm, table_sp, xv, yv, iv, *bufs_sems):
        bufs = list(bufs_sems[:6])
        sems_g = list(bufs_sems[6:12])
        sems_w = list(bufs_sems[12:18])
        sid = lax.axis_index("s")
        wid = sid * NC + lax.axis_index("c")
        base = wid * R
        half = NW * W

        @pl.when(sid == 0)
        def _():
            pltpu.sync_copy(table_hbm, table_sp)

        pltpu.sync_copy(cxy_hbm.at[pl.ds(wid * W, W)], xv)
        pltpu.sync_copy(cxy_hbm.at[pl.ds(half + wid * W, W)], yv)
        plsc.subcore_barrier()

        lane2 = 2 * lax.iota(jnp.int32, L)
        QS = SB * CR // 32

        def compute_chunk(s):
            b = s % NB
            ivb = iv.at[b]

            def cbody(q, _):
                row = (q % QS) // 4
                colbase = 32 * (q % 4)
                rows_i = jnp.broadcast_to(row, (L,))
                cols_x = lane2 + colbase
                x = xv[pl.ds(q * L, L)]
                x = jnp.minimum(jnp.maximum(x, jnp.float32(0.0)),
                                jnp.float32(n_rows_x - 1))
                x = (x + MAGIC) - MAGIC
                plsc.store_scatter(ivb, [rows_i, cols_x], x.astype(jnp.int32))
                y = yv[pl.ds(q * L, L)]
                y = jnp.minimum(jnp.maximum(y, jnp.float32(0.0)),
                                jnp.float32(n_rows_y - 1))
                y = (y + MAGIC) - MAGIC
                plsc.store_scatter(ivb, [rows_i, cols_x + 1],
                                   y.astype(jnp.int32) + jnp.int32(n_rows_x))
                return 0

            lax.fori_loop(s * QS, (s + 1) * QS, cbody, 0)

        def fire_gathers(s):
            b = s % NB
            ds_ = []
            for j in range(SB):
                ds_.append(pltpu.async_copy(
                    table_sp.at[iv.at[b, j]],
                    bufs[b].at[pl.ds(j * CR, CR)],
                    sems_g[b]))
            return ds_

        def fire_write(s):
            b = s % NB
            return pltpu.async_copy(
                bufs[b], out_hbm.at[pl.ds(base + s * SB * CR, SB * CR)],
                sems_w[b])

        gd = {}
        wd = {}
        for s in range(S):
            compute_chunk(s)
            if s >= NB:
                wd[s - NB].wait()
            gd[s] = fire_gathers(s)
            if s >= 1:
                for d in gd.pop(s - 1):
                    d.wait()
                wd[s - 1] = fire_write(s - 1)
        for d in gd.pop(S - 1):
            d.wait()
        wd[S - 1] = fire_write(S - 1)
        for s in range(S - NB, S):
            wd[s].wait()

    return k


def kernel(coords, pe_x, pe_y):
    B, N, _ = coords.shape
    Vx, D = pe_x.shape
    Vy = pe_y.shape[0]
    total_rows = B * N * 2
    table = jnp.concatenate([pe_x, pe_y], axis=0)
    cxy = jnp.moveaxis(coords, 2, 0).reshape(total_rows)
    out = _make_sc_gather(total_rows, Vx, Vy)(table, cxy)
    return out.reshape(B, N, 2 * D)


# speedup vs baseline: 1.0104x; 1.0104x over previous
import functools
import jax
import jax.numpy as jnp
from jax import lax
from jax.experimental import pallas as pl
from jax.experimental.pallas import tpu as pltpu
from jax.experimental.pallas import tpu_sc as plsc


def _make_sc_gather(total_rows, n_rows_x, n_rows_y):
    info = plsc.get_sparse_core_info()
    NC, NS, L = info.num_cores, info.num_subcores, info.num_lanes
    NW = NC * NS
    R = total_rows // NW
    W = R // 2
    CR = 128
    G = R // CR
    SB = 4
    S = G // SB
    NB = 3
    MAGIC = jnp.float32(1.5 * (2 ** 23))

    mesh = plsc.VectorSubcoreMesh(core_axis_name="c", subcore_axis_name="s")

    @functools.partial(
        pl.kernel,
        mesh=mesh,
        compiler_params=pltpu.CompilerParams(use_tc_tiling_on_sc=False,
                                             needs_layout_passes=False),
        out_type=jax.ShapeDtypeStruct((total_rows, 64), jnp.float32),
        scratch_types=[
            pltpu.VMEM_SHARED((1024, 64), jnp.float32),
            pltpu.VMEM((W,), jnp.float32),
            pltpu.VMEM((W,), jnp.float32),
            pltpu.VMEM((NB, SB, CR), jnp.int32),
            pltpu.VMEM((SB * CR, 64), jnp.float32),
            pltpu.VMEM((SB * CR, 64), jnp.float32),
            pltpu.VMEM((SB * CR, 64), jnp.float32),
            pltpu.SemaphoreType.DMA,
            pltpu.SemaphoreType.DMA,
            pltpu.SemaphoreType.DMA,
            pltpu.SemaphoreType.DMA,
            pltpu.SemaphoreType.DMA,
            pltpu.SemaphoreType.DMA,
        ],
    )
    def k(table_hbm, cxy_hbm, out_hbm, table_sp, xv, yv, iv,
          rows0, rows1, rows2, sg0, sg1, sg2, sw0, sw1, sw2):
        sid = lax.axis_index("s")
        wid = sid * NC + lax.axis_index("c")
        base = wid * R
        half = NW * W

        @pl.when(sid == 0)
        def _():
            pltpu.sync_copy(table_hbm, table_sp)

        pltpu.sync_copy(cxy_hbm.at[pl.ds(wid * W, W)], xv)
        pltpu.sync_copy(cxy_hbm.at[pl.ds(half + wid * W, W)], yv)
        plsc.subcore_barrier()

        lane2 = 2 * lax.iota(jnp.int32, L)
        QS = SB * CR // 32

        def compute_chunk(s):
            b = s % NB
            ivb = iv.at[b]

            def cbody(q, _):
                row = (q % QS) // 4
                colbase = 32 * (q % 4)
                rows_i = jnp.broadcast_to(row, (L,))
                cols_x = lane2 + colbase
                x = xv[pl.ds(q * L, L)]
                x = jnp.minimum(jnp.maximum(x, jnp.float32(0.0)),
                                jnp.float32(n_rows_x - 1))
                x = (x + MAGIC) - MAGIC
                plsc.store_scatter(ivb, [rows_i, cols_x], x.astype(jnp.int32))
                y = yv[pl.ds(q * L, L)]
                y = jnp.minimum(jnp.maximum(y, jnp.float32(0.0)),
                                jnp.float32(n_rows_y - 1))
                y = (y + MAGIC) - MAGIC
                plsc.store_scatter(ivb, [rows_i, cols_x + 1],
                                   y.astype(jnp.int32) + jnp.int32(n_rows_x))
                return 0

            lax.fori_loop(s * QS, (s + 1) * QS, cbody, 0)

        bufs = [rows0, rows1, rows2]
        sems_g = [sg0, sg1, sg2]
        sems_w = [sw0, sw1, sw2]

        def fire_gathers(s):
            b = s % NB
            ds_ = []
            for j in range(SB):
                ds_.append(pltpu.async_copy(
                    table_sp.at[iv.at[b, j]],
                    bufs[b].at[pl.ds(j * CR, CR)],
                    sems_g[b]))
            return ds_

        def fire_write(s):
            b = s % NB
            return pltpu.async_copy(
                bufs[b], out_hbm.at[pl.ds(base + s * SB * CR, SB * CR)],
                sems_w[b])

        gd = {}
        wd = {}
        for s in range(S):
            compute_chunk(s)
            if s >= NB:
                wd[s - NB].wait()
            gd[s] = fire_gathers(s)
            if s >= 1:
                for d in gd.pop(s - 1):
                    d.wait()
                wd[s - 1] = fire_write(s - 1)
        for d in gd.pop(S - 1):
            d.wait()
        wd[S - 1] = fire_write(S - 1)
        for s in range(S - NB, S):
            wd[s].wait()

    return k


def kernel(coords, pe_x, pe_y):
    B, N, _ = coords.shape
    Vx, D = pe_x.shape
    Vy = pe_y.shape[0]
    total_rows = B * N * 2
    table = jnp.concatenate([pe_x, pe_y], axis=0)
    cxy = jnp.moveaxis(coords, 2, 0).reshape(total_rows)
    out = _make_sc_gather(total_rows, Vx, Vy)(table, cxy)
    return out.reshape(B, N, 2 * D)
